# baseline (speedup 1.0000x reference)
import functools

import jax
import jax.numpy as jnp
import numpy as np
from jax import lax
from jax.experimental import pallas as pl
from jax.experimental.pallas import tpu as pltpu
from jax.experimental.pallas import tpu_sc as plsc

_L = 16
_UNROLL = 16
_NBINS = 65536
_ROWS = _NBINS // 128
_PADW = 256
_BLK = 1024


def _quantile_index_weights(n: int):
    counts = np.float32(n)
    q = np.float32(0.5) * (counts - np.float32(1.0))
    low = np.floor(q)
    high = np.ceil(q)
    hw = np.float32(q - low)
    lw = np.float32(np.float32(1.0) - hw)
    k_lo = min(max(int(low), 0), n - 1)
    k_hi = min(max(int(high), 0), n - 1)
    return k_lo, k_hi, float(lw), float(hw)


def _pick_chunk(per_w: int) -> int:
    for c in (7168, 6144, 5120, 4096, 3584, 3072, 2048, 1024, 512, 256, 128, 64, 32, 16):
        if per_w % c == 0 and (per_w // c) % 2 == 0:
            return c
    raise ValueError(f"no chunking for per-worker size {per_w}")


def _absdiff_body(p_ref, t_ref, o_ref, *, w):
    d = jnp.abs(p_ref[...] - t_ref[...])
    pad = jnp.zeros((d.shape[0], _PADW - w), jnp.float32)
    o_ref[...] = jnp.concatenate([d, pad], axis=1)


@functools.lru_cache(maxsize=None)
def _make_sc_kernels(n_pad: int):
    mesh = plsc.VectorSubcoreMesh(core_axis_name="c", subcore_axis_name="s")
    nc, ns = mesh.num_cores, mesh.num_subcores
    nw = nc * ns
    assert n_pad % (nw * _L) == 0, n_pad
    per_w = n_pad // nw
    chunk = _pick_chunk(per_w)
    nchunk = per_w // chunk
    npair = nchunk // 2
    nvec = chunk // _L

    def _start(d_hbm, buf, sem, base, g):
        off = base + g * chunk
        pltpu.make_async_copy(d_hbm.at[pl.ds(off, chunk)], buf, sem).start()

    def _wait(d_hbm, buf, sem):
        pltpu.make_async_copy(d_hbm.at[pl.ds(0, chunk)], buf, sem).wait()

    def _zero_hist(hist):
        def zb(i, c):
            hist[pl.ds(i * _L, _L)] = jnp.zeros((_L,), jnp.int32)
            return c

        lax.fori_loop(0, _NBINS // _L, zb, 0)

    stream_scratch = [
        pltpu.VMEM((chunk,), jnp.float32),
        pltpu.VMEM((chunk,), jnp.float32),
        pltpu.VMEM((_NBINS,), jnp.int32),
    ]
    sc_params = pltpu.CompilerParams(needs_layout_passes=False)

    @functools.partial(
        pl.kernel,
        out_type=jax.ShapeDtypeStruct((nw, _NBINS), jnp.int32),
        mesh=mesh,
        compiler_params=sc_params,
        scratch_types=stream_scratch
        + [pltpu.SemaphoreType.DMA, pltpu.SemaphoreType.DMA],
    )
    def pass1(d_hbm, hist_out, d0, d1, hist, sem0, sem1):
        wid = lax.axis_index("s") * nc + lax.axis_index("c")
        base = wid * per_w
        _start(d_hbm, d0, sem0, base, 0)
        _start(d_hbm, d1, sem1, base, 1)
        _zero_hist(hist)

        ones = jnp.ones((_L,), jnp.int32)

        def process(dbuf):
            @plsc.parallel_loop(0, nvec, unroll=_UNROLL)
            def _(j):
                d = dbuf[pl.ds(j * _L, _L)]
                bits = plsc.bitcast(d, jnp.int32)
                bn = lax.shift_right_logical(bits, 16)
                plsc.addupdate_scatter(hist, [bn], ones)

        def pair(i, c):
            _wait(d_hbm, d0, sem0)
            process(d0)

            @pl.when(i + 1 < npair)
            def _():
                _start(d_hbm, d0, sem0, base, 2 * i + 2)

            _wait(d_hbm, d1, sem1)
            process(d1)

            @pl.when(i + 1 < npair)
            def _():
                _start(d_hbm, d1, sem1, base, 2 * i + 3)

            return c

        lax.fori_loop(0, npair, pair, 0)
        pltpu.sync_copy(hist, hist_out.at[wid])

    @functools.partial(
        pl.kernel,
        out_type=(
            jax.ShapeDtypeStruct((nw, _NBINS), jnp.int32),
            jax.ShapeDtypeStruct((nw, 2 * _L), jnp.float32),
        ),
        mesh=mesh,
        compiler_params=sc_params,
        scratch_types=stream_scratch
        + [
            pltpu.VMEM((_L,), jnp.int32),
            pltpu.VMEM((2 * _L,), jnp.float32),
            pltpu.SemaphoreType.DMA,
            pltpu.SemaphoreType.DMA,
        ],
    )
    def pass2(d_hbm, bv_hbm, hist_out, stats_out, d0, d1, hist, pv, sv, sem0, sem1):
        wid = lax.axis_index("s") * nc + lax.axis_index("c")
        base = wid * per_w
        _start(d_hbm, d0, sem0, base, 0)
        _start(d_hbm, d1, sem1, base, 1)
        pltpu.sync_copy(bv_hbm, pv)
        bv = pv[...]
        _zero_hist(hist)

        ones = jnp.ones((_L,), jnp.int32)

        def process(dbuf, carry):
            def vb(j, c):
                sacc, macc = c
                d = dbuf[pl.ds(j * _L, _L)]
                bits = plsc.bitcast(d, jnp.int32)
                bn = lax.shift_right_logical(bits, 16)
                low = jnp.bitwise_and(bits, jnp.int32(0xFFFF))
                is_b = bn == bv
                above = bn > bv
                plsc.addupdate_scatter(hist, [low], ones, mask=is_b)
                sacc = sacc + jnp.where(above, d, jnp.float32(0.0))
                macc = jnp.minimum(macc, jnp.where(above, d, jnp.float32(np.inf)))
                return sacc, macc

            return plsc.parallel_loop(0, nvec, unroll=_UNROLL, carry=carry)(vb)

        def pair(i, carry):
            _wait(d_hbm, d0, sem0)
            carry = process(d0, carry)

            @pl.when(i + 1 < npair)
            def _():
                _start(d_hbm, d0, sem0, base, 2 * i + 2)

            _wait(d_hbm, d1, sem1)
            carry = process(d1, carry)

            @pl.when(i + 1 < npair)
            def _():
                _start(d_hbm, d1, sem1, base, 2 * i + 3)

            return carry

        carry0 = (
            jnp.zeros((_L,), jnp.float32),
            jnp.full((_L,), np.inf, jnp.float32),
        )
        sacc, macc = lax.fori_loop(0, npair, pair, carry0)
        sv[pl.ds(0, _L)] = sacc
        sv[pl.ds(_L, _L)] = macc
        pltpu.sync_copy(hist, hist_out.at[wid])
        pltpu.sync_copy(sv, stats_out.at[wid])

    return pass1, pass2, nw


def _merge_rows(h, nw):
    m = h[0:_ROWS, :]
    for w in range(1, nw):
        m = m + h[w * _ROWS : (w + 1) * _ROWS, :]
    return m


def _rank_search(m, flat, rank):
    base = jnp.int32(0)
    cb = jnp.int32(0)
    for bit in reversed(range(16)):
        mid = base + jnp.int32(1 << bit)
        seg = jnp.sum(jnp.where((flat >= base) & (flat < mid), m, jnp.int32(0)))
        c = cb + seg
        take = c <= rank
        base = jnp.where(take, mid, base)
        cb = jnp.where(take, c, cb)
    return base, cb


def _iota_flat():
    r = lax.broadcasted_iota(jnp.int32, (_ROWS, 128), 0)
    l = lax.broadcasted_iota(jnp.int32, (_ROWS, 128), 1)
    return r * jnp.int32(128) + l


def _sel1_body(h_ref, b_ref, cb_ref, *, nw, k_lo):
    m = _merge_rows(h_ref[...], nw)
    flat = _iota_flat()
    b, cb = _rank_search(m, flat, jnp.int32(k_lo))
    b_ref[0, 0] = b
    cb_ref[0, 0] = cb


def _sel2_body(
    h_ref, st_ref, b_ref, cb_ref, out_ref, *, nw, n, k_lo, k_hi, w_lo, w_hi
):
    m = _merge_rows(h_ref[...], nw)
    st = st_ref[...]
    s_above = jnp.sum(st[:, 0:_L])
    min_above = jnp.min(st[:, _L : 2 * _L])
    bbin = b_ref[0, 0]
    cbelow = cb_ref[0, 0]
    flat = _iota_flat()
    hi_bits = lax.shift_left(bbin, jnp.int32(16))

    j0 = jnp.int32(k_lo) - cbelow
    l0, _ = _rank_search(m, flat, j0)
    v_lo = lax.bitcast_convert_type(jnp.bitwise_or(hi_bits, l0), jnp.float32)
    if k_hi == k_lo:
        v_hi = v_lo
    else:
        cnt_b = jnp.sum(m)
        j1 = jnp.int32(k_hi) - cbelow
        l1, _ = _rank_search(m, flat, j1)
        v_in = lax.bitcast_convert_type(jnp.bitwise_or(hi_bits, l1), jnp.float32)
        v_hi = jnp.where(j1 < cnt_b, v_in, min_above)

    thr = jnp.float32(w_lo) * v_lo + jnp.float32(w_hi) * v_hi

    vals = lax.bitcast_convert_type(jnp.bitwise_or(hi_bits, flat), jnp.float32)
    bsum = jnp.sum(
        jnp.where(vals >= thr, m.astype(jnp.float32) * vals, jnp.float32(0.0))
    )
    out_ref[0, 0] = (s_above + bsum) / jnp.float32(n)


def kernel(pred, target):
    n = int(np.prod(pred.shape))
    w = pred.shape[-1]
    assert w <= _PADW and n % w == 0
    rows = n // w
    assert rows % _BLK == 0
    n_pad = rows * _PADW
    n_extra = n_pad - n

    k_lo, k_hi, w_lo, w_hi = _quantile_index_weights(n)
    pass1, pass2, nw = _make_sc_kernels(n_pad)

    p2 = pred.reshape(rows, w)
    t2 = target.reshape(rows, w)

    absdiff = pl.pallas_call(
        functools.partial(_absdiff_body, w=w),
        grid=(rows // _BLK,),
        in_specs=[
            pl.BlockSpec((_BLK, w), lambda i: (i, 0)),
            pl.BlockSpec((_BLK, w), lambda i: (i, 0)),
        ],
        out_specs=pl.BlockSpec((_BLK, _PADW), lambda i: (i, 0)),
        out_shape=jax.ShapeDtypeStruct((rows, _PADW), jnp.float32),
    )
    d = absdiff(p2, t2).reshape(-1)

    hist1 = pass1(d)
    _smem = pl.BlockSpec(memory_space=pltpu.SMEM)
    _vmem = pl.BlockSpec(memory_space=pltpu.VMEM)
    sel1 = pl.pallas_call(
        functools.partial(_sel1_body, nw=nw, k_lo=k_lo + n_extra),
        in_specs=[_vmem],
        out_specs=(_smem, _smem),
        out_shape=(
            jax.ShapeDtypeStruct((1, 1), jnp.int32),
            jax.ShapeDtypeStruct((1, 1), jnp.int32),
        ),
    )
    b, cb = sel1(hist1.reshape(nw * _ROWS, 128))

    bvec = jnp.broadcast_to(b[0, 0], (_L,)).astype(jnp.int32)
    hist2, stats = pass2(d, bvec)

    sel2 = pl.pallas_call(
        functools.partial(
            _sel2_body,
            nw=nw,
            n=n,
            k_lo=k_lo + n_extra,
            k_hi=k_hi + n_extra,
            w_lo=w_lo,
            w_hi=w_hi,
        ),
        in_specs=[_vmem, _vmem, _smem, _smem],
        out_specs=_smem,
        out_shape=jax.ShapeDtypeStruct((1, 1), jnp.float32),
    )
    loss = sel2(hist2.reshape(nw * _ROWS, 128), stats, b, cb)
    return loss.reshape(())


# speedup vs baseline: 1.1848x; 1.1848x over previous
import functools

import jax
import jax.numpy as jnp
import numpy as np
from jax import lax
from jax.experimental import pallas as pl
from jax.experimental.pallas import tpu as pltpu
from jax.experimental.pallas import tpu_sc as plsc

_L = 16
_UNROLL = 8
_NBINS = 65536
_ROWS = _NBINS // 128
_PADW = 256
_BLK = 1024


def _quantile_index_weights(n: int):
    counts = np.float32(n)
    q = np.float32(0.5) * (counts - np.float32(1.0))
    low = np.floor(q)
    high = np.ceil(q)
    hw = np.float32(q - low)
    lw = np.float32(np.float32(1.0) - hw)
    k_lo = min(max(int(low), 0), n - 1)
    k_hi = min(max(int(high), 0), n - 1)
    return k_lo, k_hi, float(lw), float(hw)


def _pick_chunk(per_w: int) -> int:
    for c in (7168, 6144, 5120, 4096, 3584, 3072, 2048, 1024, 512, 256, 128, 64, 32, 16):
        if per_w % c == 0 and (per_w // c) % 2 == 0:
            return c
    raise ValueError(f"no chunking for per-worker size {per_w}")


def _absdiff_body(p_ref, t_ref, o_ref, *, w):
    d = jnp.abs(p_ref[...] - t_ref[...])
    pad = jnp.zeros((d.shape[0], _PADW - w), jnp.float32)
    o_ref[...] = jnp.concatenate([d, pad], axis=1)


@functools.lru_cache(maxsize=None)
def _make_sc_kernels(total_rows: int):
    mesh = plsc.VectorSubcoreMesh(core_axis_name="c", subcore_axis_name="s")
    nc, ns = mesh.num_cores, mesh.num_subcores
    nw = nc * ns
    assert total_rows % (nw * 8) == 0, total_rows
    rows_w = total_rows // nw
    rchunk = 32
    assert rows_w % rchunk == 0 and (rows_w // rchunk) % 2 == 0
    nchunk = rows_w // rchunk
    npair = nchunk // 2
    chunk = rchunk * _PADW
    nvec = chunk // _L
    vec_per_row = _PADW // _L

    def _start(d_hbm, buf, sem, base, g):
        off = base + g * rchunk
        pltpu.make_async_copy(d_hbm.at[pl.ds(off, rchunk), :], buf, sem).start()

    def _wait(d_hbm, buf, sem):
        pltpu.make_async_copy(d_hbm.at[pl.ds(0, rchunk), :], buf, sem).wait()

    def _zero_hist(hist):
        def zb(i, c):
            hist[pl.ds(i * _L, _L)] = jnp.zeros((_L,), jnp.int32)
            return c

        lax.fori_loop(0, _NBINS // _L, zb, 0)

    def _vec(dbuf, j):
        r = lax.shift_right_logical(j, 4)
        c = jnp.bitwise_and(j, jnp.int32(vec_per_row - 1)) * _L
        return dbuf[r, pl.ds(c, _L)]

    stream_scratch = [
        pltpu.VMEM((rchunk, _PADW), jnp.float32),
        pltpu.VMEM((rchunk, _PADW), jnp.float32),
        pltpu.VMEM((_NBINS,), jnp.int32),
    ]
    sc_params = pltpu.CompilerParams(
        needs_layout_passes=False, use_tc_tiling_on_sc=True
    )

    @functools.partial(
        pl.kernel,
        out_type=jax.ShapeDtypeStruct((nw, _NBINS), jnp.int32),
        mesh=mesh,
        compiler_params=sc_params,
        scratch_types=stream_scratch
        + [pltpu.SemaphoreType.DMA, pltpu.SemaphoreType.DMA],
    )
    def pass1(d_hbm, hist_out, d0, d1, hist, sem0, sem1):
        wid = lax.axis_index("s") * nc + lax.axis_index("c")
        base = wid * rows_w
        _start(d_hbm, d0, sem0, base, 0)
        _start(d_hbm, d1, sem1, base, 1)
        _zero_hist(hist)

        ones = jnp.ones((_L,), jnp.int32)

        def process(dbuf):
            @plsc.parallel_loop(0, nvec, unroll=_UNROLL)
            def _(j):
                d = _vec(dbuf, j)
                bits = plsc.bitcast(d, jnp.int32)
                bn = lax.shift_right_logical(bits, 16)
                plsc.addupdate_scatter(hist, [bn], ones)

        def pair(i, c):
            _wait(d_hbm, d0, sem0)
            process(d0)

            @pl.when(i + 1 < npair)
            def _():
                _start(d_hbm, d0, sem0, base, 2 * i + 2)

            _wait(d_hbm, d1, sem1)
            process(d1)

            @pl.when(i + 1 < npair)
            def _():
                _start(d_hbm, d1, sem1, base, 2 * i + 3)

            return c

        lax.fori_loop(0, npair, pair, 0)
        pltpu.sync_copy(hist, hist_out.at[wid])

    @functools.partial(
        pl.kernel,
        out_type=(
            jax.ShapeDtypeStruct((nw, _NBINS), jnp.int32),
            jax.ShapeDtypeStruct((nw, 2 * _L), jnp.float32),
        ),
        mesh=mesh,
        compiler_params=sc_params,
        scratch_types=stream_scratch
        + [
            pltpu.VMEM((_L,), jnp.int32),
            pltpu.VMEM((2 * _L,), jnp.float32),
            pltpu.SemaphoreType.DMA,
            pltpu.SemaphoreType.DMA,
        ],
    )
    def pass2(d_hbm, bv_hbm, hist_out, stats_out, d0, d1, hist, pv, sv, sem0, sem1):
        wid = lax.axis_index("s") * nc + lax.axis_index("c")
        base = wid * rows_w
        _start(d_hbm, d0, sem0, base, 0)
        _start(d_hbm, d1, sem1, base, 1)
        pltpu.sync_copy(bv_hbm, pv)
        bv = pv[...]
        _zero_hist(hist)

        ones = jnp.ones((_L,), jnp.int32)

        def process(dbuf, carry):
            def vb(j, c):
                sacc, macc = c
                d = _vec(dbuf, j)
                bits = plsc.bitcast(d, jnp.int32)
                bn = lax.shift_right_logical(bits, 16)
                low = jnp.bitwise_and(bits, jnp.int32(0xFFFF))
                is_b = bn == bv
                above = bn > bv
                plsc.addupdate_scatter(hist, [low], ones, mask=is_b)
                sacc = sacc + jnp.where(above, d, jnp.float32(0.0))
                macc = jnp.minimum(macc, jnp.where(above, d, jnp.float32(np.inf)))
                return sacc, macc

            return plsc.parallel_loop(0, nvec, unroll=_UNROLL, carry=carry)(vb)

        def pair(i, carry):
            _wait(d_hbm, d0, sem0)
            carry = process(d0, carry)

            @pl.when(i + 1 < npair)
            def _():
                _start(d_hbm, d0, sem0, base, 2 * i + 2)

            _wait(d_hbm, d1, sem1)
            carry = process(d1, carry)

            @pl.when(i + 1 < npair)
            def _():
                _start(d_hbm, d1, sem1, base, 2 * i + 3)

            return carry

        carry0 = (
            jnp.zeros((_L,), jnp.float32),
            jnp.full((_L,), np.inf, jnp.float32),
        )
        sacc, macc = lax.fori_loop(0, npair, pair, carry0)
        sv[pl.ds(0, _L)] = sacc
        sv[pl.ds(_L, _L)] = macc
        pltpu.sync_copy(hist, hist_out.at[wid])
        pltpu.sync_copy(sv, stats_out.at[wid])

    return pass1, pass2, nw


def _merge_rows(h, nw):
    m = h[0:_ROWS, :]
    for w in range(1, nw):
        m = m + h[w * _ROWS : (w + 1) * _ROWS, :]
    return m


def _rank_search(m, flat, rank):
    base = jnp.int32(0)
    cb = jnp.int32(0)
    for bit in reversed(range(16)):
        mid = base + jnp.int32(1 << bit)
        seg = jnp.sum(jnp.where((flat >= base) & (flat < mid), m, jnp.int32(0)))
        c = cb + seg
        take = c <= rank
        base = jnp.where(take, mid, base)
        cb = jnp.where(take, c, cb)
    return base, cb


def _iota_flat():
    r = lax.broadcasted_iota(jnp.int32, (_ROWS, 128), 0)
    l = lax.broadcasted_iota(jnp.int32, (_ROWS, 128), 1)
    return r * jnp.int32(128) + l


def _sel1_body(h_ref, b_ref, cb_ref, *, nw, k_lo):
    m = _merge_rows(h_ref[...], nw)
    flat = _iota_flat()
    b, cb = _rank_search(m, flat, jnp.int32(k_lo))
    b_ref[0, 0] = b
    cb_ref[0, 0] = cb


def _sel2_body(
    h_ref, st_ref, b_ref, cb_ref, out_ref, *, nw, n, k_lo, k_hi, w_lo, w_hi
):
    m = _merge_rows(h_ref[...], nw)
    st = st_ref[...]
    s_above = jnp.sum(st[:, 0:_L])
    min_above = jnp.min(st[:, _L : 2 * _L])
    bbin = b_ref[0, 0]
    cbelow = cb_ref[0, 0]
    flat = _iota_flat()
    hi_bits = lax.shift_left(bbin, jnp.int32(16))

    j0 = jnp.int32(k_lo) - cbelow
    l0, _ = _rank_search(m, flat, j0)
    v_lo = lax.bitcast_convert_type(jnp.bitwise_or(hi_bits, l0), jnp.float32)
    if k_hi == k_lo:
        v_hi = v_lo
    else:
        cnt_b = jnp.sum(m)
        j1 = jnp.int32(k_hi) - cbelow
        l1, _ = _rank_search(m, flat, j1)
        v_in = lax.bitcast_convert_type(jnp.bitwise_or(hi_bits, l1), jnp.float32)
        v_hi = jnp.where(j1 < cnt_b, v_in, min_above)

    thr = jnp.float32(w_lo) * v_lo + jnp.float32(w_hi) * v_hi

    vals = lax.bitcast_convert_type(jnp.bitwise_or(hi_bits, flat), jnp.float32)
    bsum = jnp.sum(
        jnp.where(vals >= thr, m.astype(jnp.float32) * vals, jnp.float32(0.0))
    )
    out_ref[0, 0] = (s_above + bsum) / jnp.float32(n)


def kernel(pred, target):
    n = int(np.prod(pred.shape))
    w = pred.shape[-1]
    assert w <= _PADW and n % w == 0
    rows = n // w
    assert rows % _BLK == 0
    n_pad = rows * _PADW
    n_extra = n_pad - n

    k_lo, k_hi, w_lo, w_hi = _quantile_index_weights(n)
    pass1, pass2, nw = _make_sc_kernels(rows)

    p2 = pred.reshape(rows, w)
    t2 = target.reshape(rows, w)

    absdiff = pl.pallas_call(
        functools.partial(_absdiff_body, w=w),
        grid=(rows // _BLK,),
        in_specs=[
            pl.BlockSpec((_BLK, w), lambda i: (i, 0)),
            pl.BlockSpec((_BLK, w), lambda i: (i, 0)),
        ],
        out_specs=pl.BlockSpec((_BLK, _PADW), lambda i: (i, 0)),
        out_shape=jax.ShapeDtypeStruct((rows, _PADW), jnp.float32),
    )
    d = absdiff(p2, t2)

    hist1 = pass1(d)
    _smem = pl.BlockSpec(memory_space=pltpu.SMEM)
    _vmem = pl.BlockSpec(memory_space=pltpu.VMEM)
    sel1 = pl.pallas_call(
        functools.partial(_sel1_body, nw=nw, k_lo=k_lo + n_extra),
        in_specs=[_vmem],
        out_specs=(_smem, _smem),
        out_shape=(
            jax.ShapeDtypeStruct((1, 1), jnp.int32),
            jax.ShapeDtypeStruct((1, 1), jnp.int32),
        ),
    )
    b, cb = sel1(hist1.reshape(nw * _ROWS, 128))

    bvec = jnp.broadcast_to(b[0, 0], (_L,)).astype(jnp.int32)
    hist2, stats = pass2(d, bvec)

    sel2 = pl.pallas_call(
        functools.partial(
            _sel2_body,
            nw=nw,
            n=n,
            k_lo=k_lo + n_extra,
            k_hi=k_hi + n_extra,
            w_lo=w_lo,
            w_hi=w_hi,
        ),
        in_specs=[_vmem, _vmem, _smem, _smem],
        out_specs=_smem,
        out_shape=jax.ShapeDtypeStruct((1, 1), jnp.float32),
    )
    loss = sel2(hist2.reshape(nw * _ROWS, 128), stats, b, cb)
    return loss.reshape(())
